# Initial kernel scaffold; baseline (speedup 1.0000x reference)
import jax
import jax.numpy as jnp
from jax.experimental import pallas as pl


def kernel(x, stem_w, fc1_w, fc2_w, fc3_w, ff1_w, ff1_b, ff2_w, ff2_b, head1_w, head2_w, head2_b):
    raise NotImplementedError("write your pallas kernel here")


# speedup vs baseline: 3.8940x; 3.8940x over previous
import functools

import jax
import jax.numpy as jnp
import numpy as np
from jax import lax
from jax.experimental import pallas as pl
from jax.experimental.pallas import tpu as pltpu

_C = 192
_N = 196
_K = 16
_B = 8
_PATCH = 16


def _stem_body(p_ref, w_ref, o_ref):
    h = jnp.dot(p_ref[...], w_ref[...], preferred_element_type=jnp.float32)
    inv = jnp.float32(1.0 / (_B * _N))
    mu = jnp.sum(h, axis=0, keepdims=True) * inv
    var = jnp.sum((h - mu) ** 2, axis=0, keepdims=True) * inv
    o_ref[...] = (h - mu) / jnp.sqrt(var + 1e-5)


def _block_body(f_ref, fc1_ref, fc2_ref, fc3_ref,
                ff1_ref, ff1b_ref, ff2_ref, ff2b_ref, o_ref):
    ft = f_ref[0]
    fb = ft.astype(jnp.bfloat16)
    y = jnp.dot(fc1_ref[...], fb,
                preferred_element_type=jnp.float32)
    sq = jnp.sum(y * y, axis=0, keepdims=True)
    g = lax.dot_general(y, y, (((0,), (0,)), ((), ())),
                        preferred_element_type=jnp.float32)
    sq_col = jnp.transpose(sq)
    d = (sq_col + sq) - 2.0 * g
    iota = lax.broadcasted_iota(jnp.int32, (_N, _N), 0)
    work = d
    rel = jnp.full((_C, _N), -3.0e38, jnp.float32)
    for _ in range(_K):
        m = jnp.min(work, axis=0, keepdims=True)
        is_min = work == m
        jsel = jnp.min(jnp.where(is_min, iota, jnp.int32(2**30)), axis=0,
                       keepdims=True)
        onehot = iota == jsel
        work = jnp.where(onehot, jnp.float32(3.0e38), work)
        nb = jnp.dot(y, onehot.astype(jnp.float32),
                     preferred_element_type=jnp.float32,
                     precision=lax.Precision.HIGHEST)
        rel = jnp.maximum(rel, nb)
    relb = (rel - y).astype(jnp.bfloat16)
    z = jnp.concatenate([y.astype(jnp.bfloat16), relb], axis=0)
    z = jnp.dot(fc2_ref[...], z, preferred_element_type=jnp.float32)
    z = jax.nn.gelu(z)
    z = jnp.dot(fc3_ref[...], z.astype(jnp.bfloat16),
                preferred_element_type=jnp.float32)
    gres = ft + z
    h = jnp.dot(ff1_ref[...], gres,
                preferred_element_type=jnp.float32) + ff1b_ref[...]
    h = jax.nn.gelu(h)
    h = jnp.dot(ff2_ref[...], h.astype(jnp.bfloat16),
                preferred_element_type=jnp.float32) + ff2b_ref[...]
    o_ref[0] = jax.nn.gelu(gres + h)


def _head1_body(x_ref, w_ref, o_ref):
    @pl.when(pl.program_id(0) == 0)
    def _():
        o_ref[...] = jnp.zeros_like(o_ref)

    o_ref[...] += jnp.dot(x_ref[...], w_ref[...],
                          preferred_element_type=jnp.float32)


def _head2_body(h_ref, w_ref, b_ref, o_ref):
    o_ref[...] = jnp.dot(jax.nn.gelu(h_ref[...]), w_ref[...],
                         preferred_element_type=jnp.float32) + b_ref[...]


def _full(shape):
    nd = len(shape)
    return pl.BlockSpec(shape, lambda *args: (0,) * nd)


def kernel(x, stem_w, fc1_w, fc2_w, fc3_w, ff1_w, ff1_b, ff2_w, ff2_b,
           head1_w, head2_w, head2_b):
    BN = _B * _N
    PD = 3 * _PATCH * _PATCH
    bf = jnp.bfloat16
    h = lax.conv_general_dilated(x, stem_w, (_PATCH, _PATCH), 'VALID',
                                 dimension_numbers=('NCHW', 'OIHW', 'NCHW'))
    mean = jnp.mean(h, axis=(0, 2, 3), keepdims=True)
    var = jnp.var(h, axis=(0, 2, 3), keepdims=True)
    h = (h - mean) / jnp.sqrt(var + 1e-5)
    feat_t = h.reshape(_B, _C, _N)

    block_call = pl.pallas_call(
        _block_body,
        grid=(_B,),
        in_specs=[
            pl.BlockSpec((1, _C, _N), lambda b: (b, 0, 0)),
            _full((_C, _C)),
            _full((_C, 2 * _C)),
            _full((_C, _C)),
            _full((4 * _C, _C)),
            _full((4 * _C, 1)),
            _full((_C, 4 * _C)),
            _full((_C, 1)),
        ],
        out_specs=pl.BlockSpec((1, _C, _N), lambda b: (b, 0, 0)),
        out_shape=jax.ShapeDtypeStruct((_B, _C, _N), jnp.float32),
    )
    for i in range(fc1_w.shape[0]):
        feat_t = block_call(
            feat_t,
            fc1_w[i].T.astype(bf),
            fc2_w[i].T.astype(bf),
            fc3_w[i].T.astype(bf),
            ff1_w[i].T.astype(bf).astype(jnp.float32),
            ff1_b[i].reshape(4 * _C, 1),
            ff2_w[i].T.astype(bf),
            ff2_b[i].reshape(_C, 1),
        )

    flat = feat_t.reshape(_B, _C * _N)
    KT = _C * _N
    NSPLIT = 14
    KC = KT // NSPLIT
    h1 = pl.pallas_call(
        _head1_body,
        grid=(NSPLIT,),
        in_specs=[
            pl.BlockSpec((_B, KC), lambda k: (0, k)),
            pl.BlockSpec((KC, 512), lambda k: (k, 0)),
        ],
        out_specs=pl.BlockSpec((_B, 512), lambda k: (0, 0)),
        out_shape=jax.ShapeDtypeStruct((_B, 512), jnp.float32),
    )(flat.astype(bf).astype(jnp.float32), head1_w)

    out = pl.pallas_call(
        _head2_body,
        out_shape=jax.ShapeDtypeStruct((_B, 1000), jnp.float32),
    )(h1, head2_w, head2_b.reshape(1, 1000))
    return out
